# baseline (device time: 25265 ns/iter reference)
import jax
import jax.numpy as jnp
from jax import lax
from jax.experimental import pallas as pl
from jax.experimental.pallas import tpu as pltpu

N_DEV = 4


def kernel(x):
    m_per, n = x.shape
    half = m_per // 2

    def body(x_hbm, out_hbm, x_v, own_v, from_l, from_r, send_sems, recv_sems, loc_sems):
        my_pos = lax.axis_index("i")
        left = lax.rem(my_pos + N_DEV - 1, N_DEV)
        right = lax.rem(my_pos + 1, N_DEV)

        def rdma(src, dst, sem, target):
            return pltpu.make_async_remote_copy(
                src_ref=src,
                dst_ref=dst,
                send_sem=send_sems.at[sem],
                recv_sem=recv_sems.at[sem],
                device_id=(target,),
                device_id_type=pl.DeviceIdType.MESH,
            )

        barrier_sem = pltpu.get_barrier_semaphore()
        for nbr in (left, right):
            pl.semaphore_signal(
                barrier_sem,
                inc=1,
                device_id=(nbr,),
                device_id_type=pl.DeviceIdType.MESH,
            )
        cp_in = pltpu.make_async_copy(x_hbm, x_v, loc_sems.at[0])
        cp_in.start()
        cp_in.wait()
        own_v[...] = x_v[...].astype(jnp.bfloat16)
        cp_own = pltpu.make_async_copy(
            own_v, out_hbm.at[pl.ds(my_pos * m_per, m_per)], loc_sems.at[1]
        )
        cp_own.start()
        pl.semaphore_wait(barrier_sem, 2)

        lo = pl.ds(0, half)
        hi = pl.ds(half, half)
        r_lo = rdma(own_v.at[lo], from_l.at[lo], 0, right)
        l_hi = rdma(own_v.at[hi], from_r.at[hi], 3, left)
        r_hi = rdma(own_v.at[hi], from_l.at[hi], 1, right)
        l_lo = rdma(own_v.at[lo], from_r.at[lo], 4, left)
        r_lo.start()
        l_hi.start()
        r_hi.start()
        l_lo.start()

        r_lo.wait_recv()
        fwd_cw = rdma(
            from_l.at[lo], out_hbm.at[pl.ds(left * m_per, half)], 2, right
        )
        fwd_cw.start()

        l_hi.wait_recv()
        fwd_ccw = rdma(
            from_r.at[hi], out_hbm.at[pl.ds(right * m_per + half, half)], 5, left
        )
        fwd_ccw.start()

        r_hi.wait_recv()
        cp_l = pltpu.make_async_copy(
            from_l, out_hbm.at[pl.ds(left * m_per, m_per)], loc_sems.at[2]
        )
        cp_l.start()
        l_lo.wait_recv()
        cp_r = pltpu.make_async_copy(
            from_r, out_hbm.at[pl.ds(right * m_per, m_per)], loc_sems.at[3]
        )
        cp_r.start()

        fwd_cw.wait_recv()
        fwd_ccw.wait_recv()

        cp_own.wait()
        cp_l.wait()
        cp_r.wait()
        for r in (r_lo, l_hi, r_hi, l_lo, fwd_cw, fwd_ccw):
            r.wait_send()

    return pl.pallas_call(
        body,
        out_shape=jax.ShapeDtypeStruct((N_DEV * m_per, n), jnp.bfloat16),
        in_specs=[pl.BlockSpec(memory_space=pl.ANY)],
        out_specs=pl.BlockSpec(memory_space=pl.ANY),
        scratch_shapes=[
            pltpu.VMEM((m_per, n), x.dtype),
            pltpu.VMEM((m_per, n), jnp.bfloat16),
            pltpu.VMEM((m_per, n), jnp.bfloat16),
            pltpu.VMEM((m_per, n), jnp.bfloat16),
            pltpu.SemaphoreType.DMA((6,)),
            pltpu.SemaphoreType.DMA((6,)),
            pltpu.SemaphoreType.DMA((4,)),
        ],
        compiler_params=pltpu.CompilerParams(collective_id=0),
    )(x)


# device time: 25232 ns/iter; 1.0013x vs baseline; 1.0013x over previous
import jax
import jax.numpy as jnp
from jax import lax
from jax.experimental import pallas as pl
from jax.experimental.pallas import tpu as pltpu

N_DEV = 4


def kernel(x):
    m_per, n = x.shape
    half = m_per // 2

    def body(x_ref, out_ref, send_sems, recv_sems):
        my_pos = lax.axis_index("i")
        left = lax.rem(my_pos + N_DEV - 1, N_DEV)
        right = lax.rem(my_pos + 1, N_DEV)

        def copy(row_start, nrows, sem, target):
            sl = out_ref.at[pl.ds(row_start, nrows)]
            return pltpu.make_async_remote_copy(
                src_ref=sl,
                dst_ref=sl,
                send_sem=send_sems.at[sem],
                recv_sem=recv_sems.at[sem],
                device_id=(target,),
                device_id_type=pl.DeviceIdType.MESH,
            )

        barrier_sem = pltpu.get_barrier_semaphore()
        for nbr in (left, right):
            pl.semaphore_signal(
                barrier_sem,
                inc=1,
                device_id=(nbr,),
                device_id_type=pl.DeviceIdType.MESH,
            )
        out_ref[pl.ds(my_pos * m_per, m_per), :] = x_ref[...].astype(jnp.bfloat16)
        pl.semaphore_wait(barrier_sem, 2)

        r_lo = copy(my_pos * m_per, half, 0, right)
        l_hi = copy(my_pos * m_per + half, half, 3, left)
        r_hi = copy(my_pos * m_per + half, half, 1, right)
        l_lo = copy(my_pos * m_per, half, 4, left)
        r_lo.start()
        l_hi.start()
        r_hi.start()
        l_lo.start()

        r_lo.wait_recv()
        fwd_cw = copy(left * m_per, half, 2, right)
        fwd_cw.start()

        l_hi.wait_recv()
        fwd_ccw = copy(right * m_per + half, half, 5, left)
        fwd_ccw.start()

        r_hi.wait_recv()
        l_lo.wait_recv()
        fwd_cw.wait_recv()
        fwd_ccw.wait_recv()

        for rdma in (r_lo, l_hi, r_hi, l_lo, fwd_cw, fwd_ccw):
            rdma.wait_send()

    return pl.pallas_call(
        body,
        out_shape=jax.ShapeDtypeStruct((N_DEV * m_per, n), jnp.bfloat16),
        in_specs=[pl.BlockSpec(memory_space=pltpu.VMEM)],
        out_specs=pl.BlockSpec(memory_space=pltpu.VMEM),
        scratch_shapes=[
            pltpu.SemaphoreType.DMA((6,)),
            pltpu.SemaphoreType.DMA((6,)),
        ],
        compiler_params=pltpu.CompilerParams(collective_id=0),
    )(x)
